# Initial kernel scaffold; baseline (speedup 1.0000x reference)
import jax
import jax.numpy as jnp
from jax.experimental import pallas as pl


def kernel(x_ingredient, x_molecule, ei_contains_src, ei_contains_dst, ei_rev_src, ei_rev_dst, ei_cooc_src, ei_cooc_dst, ei_sim_src, ei_sim_dst, params):
    raise NotImplementedError("write your pallas kernel here")


# speedup vs baseline: 11.8068x; 11.8068x over previous
import functools
import math

import jax
import jax.numpy as jnp
from jax import lax
from jax.experimental import pallas as pl
from jax.experimental.pallas import tpu as pltpu
from jax.experimental.pallas import tpu_sc as plsc

HID = 128
H = 8
C = 16
EMB = 64
L = 3
NI = 10000
NM = 50000

NC = 2
NS = 16
NW = NC * NS
ZR = 40


def _mm(x, w, b=None, relu=False, block_rows=512):
    n, kd = x.shape
    co = w.shape[1]
    if b is None:
        b = jnp.zeros((co,), jnp.float32)

    def body(x_ref, w_ref, b_ref, o_ref):
        acc = jnp.dot(x_ref[...], w_ref[...], preferred_element_type=jnp.float32)
        acc = acc + b_ref[...]
        if relu:
            acc = jnp.maximum(acc, 0.0)
        o_ref[...] = acc

    return pl.pallas_call(
        body,
        grid=(pl.cdiv(n, block_rows),),
        in_specs=[
            pl.BlockSpec((block_rows, kd), lambda i: (i, 0)),
            pl.BlockSpec((kd, co), lambda i: (0, 0)),
            pl.BlockSpec((1, co), lambda i: (0, 0)),
        ],
        out_specs=pl.BlockSpec((block_rows, co), lambda i: (i, 0)),
        out_shape=jax.ShapeDtypeStruct((n, co), jnp.float32),
    )(x, w, b.reshape(1, co))


def _sc_edge_pass(hs_view, aux_src, aux_dst, src, dst, zeros_hbm, *,
                  e_total, k, n_dst, nb, hs_mul, hs_add, es_off, ed_off,
                  block_heads, with_den):
    w_cols = nb * 16 + (8 if with_den else 0)
    n_chunks = e_total // k
    n_blk = n_dst // ZR
    mesh = plsc.VectorSubcoreMesh(core_axis_name="c", subcore_axis_name="s",
                                  num_cores=NC, num_subcores=NS)

    def body(hs_r, auxs_r, auxd_r, src_r, dst_r, zh_r, out_r,
             acc, zrows, srcb, dstb, hsix, edix, esr, edr, hsr, rowsb):
        c = lax.axis_index("c")
        s = lax.axis_index("s")
        wid = s * NC + c

        pltpu.sync_copy(zh_r, zrows)
        n_myblk = lax.div(n_blk - 1 - s, NS) + 1

        def zbody(i, carry):
            pltpu.sync_copy(zrows, acc.at[pl.ds((s + i * NS) * ZR, ZR)])
            return carry
        lax.fori_loop(0, n_myblk, zbody, 0)
        plsc.subcore_barrier()

        iot = jax.lax.iota(jnp.int32, 16)
        n_my = lax.div(n_chunks - 1 - wid, NW) + 1

        def chunk(i, carry):
            e0 = (wid + i * NW) * k
            pltpu.sync_copy(src_r.at[pl.ds(e0, k)], srcb)
            pltpu.sync_copy(dst_r.at[pl.ds(e0, k)], dstb)
            for g in range(k // 16):
                sv = srcb[pl.ds(g * 16, 16)]
                dv = dstb[pl.ds(g * 16, 16)]
                hsix[pl.ds(g * 16, 16)] = sv * hs_mul + hs_add
                srcb[pl.ds(g * 16, 16)] = sv * 2
                edix[pl.ds(g * 16, 16)] = dv * 2 + 1
            pltpu.sync_copy(auxs_r.at[srcb], esr)
            pltpu.sync_copy(auxd_r.at[edix], edr)
            pltpu.sync_copy(hs_r.at[hsix], hsr)
            for g in range(k // 16):
                rvec = g * 16 + iot
                heads = range(H) if with_den else block_heads
                wv = {}
                for h in heads:
                    e_s = plsc.load_gather(esr, [rvec, jnp.full((16,), es_off + h, jnp.int32)])
                    e_d = plsc.load_gather(edr, [rvec, jnp.full((16,), ed_off + h, jnp.int32)])
                    e = e_s + e_d
                    e = jnp.where(e > 0, e, 0.2 * e)
                    wv[h] = jnp.exp(e)
                    if with_den:
                        plsc.store_scatter(
                            rowsb,
                            [rvec, jnp.full((16,), nb * 16 + h, jnp.int32)],
                            wv[h])
                for blk in range(nb):
                    wq = wv[block_heads[blk]]
                    for ch in range(16):
                        hv = plsc.load_gather(
                            hsr,
                            [rvec, jnp.full((16,), blk, jnp.int32),
                             jnp.full((16,), ch, jnp.int32)])
                        plsc.store_scatter(
                            rowsb,
                            [rvec, jnp.full((16,), blk * 16 + ch, jnp.int32)],
                            hv * wq)
            pltpu.sync_copy(rowsb, acc.at[dstb], add=True)
            return carry
        lax.fori_loop(0, n_my, chunk, 0)
        plsc.subcore_barrier()

        def wbody(i, carry):
            b0 = (s + i * NS) * ZR
            pltpu.sync_copy(acc.at[pl.ds(b0, ZR)],
                            out_r.at[pl.ds(c * n_dst + b0, ZR)])
            return carry
        lax.fori_loop(0, n_myblk, wbody, 0)

    out = pl.kernel(
        body,
        out_type=jax.ShapeDtypeStruct((2 * n_dst, w_cols), jnp.float32),
        mesh=mesh,
        compiler_params=pltpu.CompilerParams(needs_layout_passes=False,
                                             use_tc_tiling_on_sc=False),
        scratch_types=[
            pltpu.VMEM_SHARED((n_dst, w_cols), jnp.float32),
            pltpu.VMEM((ZR, w_cols), jnp.float32),
            pltpu.VMEM((k,), jnp.int32),
            pltpu.VMEM((k,), jnp.int32),
            pltpu.VMEM((k,), jnp.int32),
            pltpu.VMEM((k,), jnp.int32),
            pltpu.VMEM((k, 16), jnp.float32),
            pltpu.VMEM((k, 16), jnp.float32),
            pltpu.VMEM((k, nb, 16), jnp.float32),
            pltpu.VMEM((k, w_cols), jnp.float32),
        ],
    )(hs_view, aux_src, aux_dst, src, dst, zeros_hbm)
    return out.reshape(2, n_dst, w_cols)


_BN_INV = 1.0 / math.sqrt(1.0 + 1e-5)


def _norm_from_passes(ts0, ts_rest, bias):
    unnorm = jnp.concatenate([ts0[:, :32]] + list(ts_rest), axis=1)
    den = ts0[:, 32:40]
    r = unnorm.shape[0]
    dinv = 1.0 / (den + 1e-16)
    dinvb = jnp.reshape(jnp.broadcast_to(dinv[:, :, None], (r, 8, 16)), (r, 128))
    return unnorm * dinvb + bias


def _epilogue_mol(con, sim, b_con, b_sim, g, bb, block_rows=400):
    def body(c0, c1, c2, c3, s0, s1, s2, s3, bc, bs, g_r, bb_r, o_ref):
        def ts(ref):
            a = ref[...]
            return a[0] + a[1]
        out_c = _norm_from_passes(ts(c0), [ts(c1), ts(c2), ts(c3)], bc[...])
        out_s = _norm_from_passes(ts(s0), [ts(s1), ts(s2), ts(s3)], bs[...])
        x = (out_c + out_s) * _BN_INV * g_r[...] + bb_r[...]
        o_ref[...] = jnp.maximum(x, 0.0)

    specs = []
    for arr in con + sim:
        w = arr.shape[2]
        specs.append(pl.BlockSpec((2, block_rows, w), lambda i: (0, i, 0)))
    for _ in range(4):
        specs.append(pl.BlockSpec((1, 128), lambda i: (0, 0)))
    return pl.pallas_call(
        body,
        grid=(NM // block_rows,),
        in_specs=specs,
        out_specs=pl.BlockSpec((block_rows, 128), lambda i: (i, 0)),
        out_shape=jax.ShapeDtypeStruct((NM, 128), jnp.float32),
    )(*con, *sim, b_con.reshape(1, 128), b_sim.reshape(1, 128),
      g.reshape(1, 128), bb.reshape(1, 128))


def _epilogue_ing(rev, cooc, b_rev, b_cooc, g, bb, block_rows=400):
    def body(r_ref, c_ref, br, bc, g_r, bb_r, o_ref):
        def one(ref, bias):
            a = ref[...]
            ts = a[0] + a[1]
            unnorm = ts[:, :128]
            den = ts[:, 128:136]
            r = unnorm.shape[0]
            dinv = 1.0 / (den + 1e-16)
            dinvb = jnp.reshape(
                jnp.broadcast_to(dinv[:, :, None], (r, 8, 16)), (r, 128))
            return unnorm * dinvb + bias
        x = one(r_ref, br[...]) + one(c_ref, bc[...])
        x = x * _BN_INV * g_r[...] + bb_r[...]
        o_ref[...] = jnp.maximum(x, 0.0)

    return pl.pallas_call(
        body,
        grid=(NI // block_rows,),
        in_specs=[
            pl.BlockSpec((2, block_rows, 136), lambda i: (0, i, 0)),
            pl.BlockSpec((2, block_rows, 136), lambda i: (0, i, 0)),
            pl.BlockSpec((1, 128), lambda i: (0, 0)),
            pl.BlockSpec((1, 128), lambda i: (0, 0)),
            pl.BlockSpec((1, 128), lambda i: (0, 0)),
            pl.BlockSpec((1, 128), lambda i: (0, 0)),
        ],
        out_specs=pl.BlockSpec((block_rows, 128), lambda i: (i, 0)),
        out_shape=jax.ShapeDtypeStruct((NI, 128), jnp.float32),
    )(rev, cooc, b_rev.reshape(1, 128), b_cooc.reshape(1, 128),
      g.reshape(1, 128), bb.reshape(1, 128))


def _fold_att(w_mat, a_vec):
    return (w_mat.reshape(HID, H, C) * a_vec[None]).sum(-1)


def kernel(x_ingredient, x_molecule, ei_contains_src, ei_contains_dst,
           ei_rev_src, ei_rev_dst, ei_cooc_src, ei_cooc_dst,
           ei_sim_src, ei_sim_dst, params):
    p = params
    cs, cd = ei_contains_src, ei_contains_dst
    rs, rd = ei_rev_src, ei_rev_dst
    os_, od_ = ei_cooc_src, ei_cooc_dst
    ss, sd = ei_sim_src, ei_sim_dst

    z40 = jnp.zeros((ZR, 40), jnp.float32)
    z32 = jnp.zeros((ZR, 32), jnp.float32)
    z136 = jnp.zeros((ZR, 136), jnp.float32)

    x_i = _mm(x_ingredient, p["proj_ing_W"], p["proj_ing_b"], relu=True)
    x_m = _mm(x_molecule, p["proj_mol_W"], p["proj_mol_b"], relu=True)

    for l in range(L):
        pre = "l%d_" % l
        ws_con, wd_con = p[pre + "con_Ws"], p[pre + "con_Wd"]
        ws_rev, wd_rev = p[pre + "rev_Ws"], p[pre + "rev_Wd"]
        ws_cooc = p[pre + "cooc_Ws"]
        ws_sim = p[pre + "sim_Ws"]

        w_i_big = jnp.concatenate([ws_con, ws_cooc], axis=1)
        w_m_big = jnp.concatenate([ws_rev, ws_sim], axis=1)
        a_i = jnp.concatenate([
            _fold_att(ws_con, p[pre + "con_as"]),
            _fold_att(ws_cooc, p[pre + "cooc_as"]),
            _fold_att(ws_cooc, p[pre + "cooc_ad"]),
            _fold_att(wd_rev, p[pre + "rev_ad"]),
        ], axis=1)
        a_m = jnp.concatenate([
            _fold_att(ws_rev, p[pre + "rev_as"]),
            _fold_att(ws_sim, p[pre + "sim_as"]),
            _fold_att(ws_sim, p[pre + "sim_ad"]),
            _fold_att(wd_con, p[pre + "con_ad"]),
        ], axis=1)

        h_i = _mm(x_i, w_i_big)
        h_m = _mm(x_m, w_m_big)
        aux_i = _mm(x_i, a_i).reshape(NI * 2, 16)
        aux_m = _mm(x_m, a_m).reshape(NM * 2, 16)

        h_i_pairs = h_i.reshape(NI * 8, 2, 16)
        h_m_pairs = h_m.reshape(NM * 8, 2, 16)
        h_i_full = h_i.reshape(NI * 2, 8, 16)
        h_m_full = h_m.reshape(NM * 2, 8, 16)

        con_passes = tuple(
            _sc_edge_pass(h_i_pairs, aux_i, aux_m, cs, cd,
                          z40 if pr == 0 else z32,
                          e_total=320000, k=32, n_dst=NM, nb=2,
                          hs_mul=8, hs_add=pr, es_off=0, ed_off=8,
                          block_heads=(2 * pr, 2 * pr + 1), with_den=(pr == 0))
            for pr in range(4))
        sim_passes = tuple(
            _sc_edge_pass(h_m_pairs, aux_m, aux_m, ss, sd,
                          z40 if pr == 0 else z32,
                          e_total=320000, k=32, n_dst=NM, nb=2,
                          hs_mul=8, hs_add=4 + pr, es_off=8, ed_off=0,
                          block_heads=(2 * pr, 2 * pr + 1), with_den=(pr == 0))
            for pr in range(4))
        rev_pass = _sc_edge_pass(h_m_full, aux_m, aux_i, rs, rd, z136,
                                 e_total=320000, k=64, n_dst=NI, nb=8,
                                 hs_mul=2, hs_add=0, es_off=0, ed_off=8,
                                 block_heads=tuple(range(8)), with_den=True)
        cooc_pass = _sc_edge_pass(h_i_full, aux_i, aux_i, os_, od_, z136,
                                  e_total=160000, k=64, n_dst=NI, nb=8,
                                  hs_mul=2, hs_add=1, es_off=8, ed_off=0,
                                  block_heads=tuple(range(8)), with_den=True)

        x_m = _epilogue_mol(con_passes, sim_passes,
                            p[pre + "con_b"], p[pre + "sim_b"],
                            p[pre + "bn_mol_g"], p[pre + "bn_mol_b"])
        x_i = _epilogue_ing(rev_pass, cooc_pass,
                            p[pre + "rev_b"], p[pre + "cooc_b"],
                            p[pre + "bn_ing_g"], p[pre + "bn_ing_b"])

    oi = _mm(x_i, p["emb_ing_W"], p["emb_ing_b"])
    om = _mm(x_m, p["emb_mol_W"], p["emb_mol_b"])
    return oi, om


# speedup vs baseline: 21.0677x; 1.7844x over previous
import functools
import math

import jax
import jax.numpy as jnp
from jax import lax
from jax.experimental import pallas as pl
from jax.experimental.pallas import tpu as pltpu
from jax.experimental.pallas import tpu_sc as plsc

HID = 128
H = 8
C = 16
EMB = 64
L = 3
NI = 10000
NM = 50000

NC = 2
NS = 16
NW = NC * NS
ZR = 40


def _mm(x, w, b=None, relu=False, block_rows=512):
    n, kd = x.shape
    co = w.shape[1]
    if b is None:
        b = jnp.zeros((co,), jnp.float32)

    def body(x_ref, w_ref, b_ref, o_ref):
        acc = jnp.dot(x_ref[...], w_ref[...], preferred_element_type=jnp.float32)
        acc = acc + b_ref[...]
        if relu:
            acc = jnp.maximum(acc, 0.0)
        o_ref[...] = acc

    return pl.pallas_call(
        body,
        grid=(pl.cdiv(n, block_rows),),
        in_specs=[
            pl.BlockSpec((block_rows, kd), lambda i: (i, 0)),
            pl.BlockSpec((kd, co), lambda i: (0, 0)),
            pl.BlockSpec((1, co), lambda i: (0, 0)),
        ],
        out_specs=pl.BlockSpec((block_rows, co), lambda i: (i, 0)),
        out_shape=jax.ShapeDtypeStruct((n, co), jnp.float32),
    )(x, w, b.reshape(1, co))


def _sc_pass(mode, operands, *, e_total, k, n_dst, zr, hs_mul=0, hs_add=0,
             es_off=0, ed_off=0, pair=0):
    w_cols = {"den": 8, "hs": 32, "full": 136}[mode]
    nb = {"den": 0, "hs": 2, "full": 8}[mode]
    n_chunks = e_total // k
    n_blk = n_dst // zr
    mesh = plsc.VectorSubcoreMesh(core_axis_name="c", subcore_axis_name="s",
                                  num_cores=NC, num_subcores=NS)

    n_in = len(operands)
    out_type = [jax.ShapeDtypeStruct((2 * n_dst, w_cols), jnp.float32)]
    if mode == "den":
        out_type.append(jax.ShapeDtypeStruct((e_total, 8), jnp.float32))

    scratch = [
        pltpu.VMEM_SHARED((n_dst, w_cols), jnp.float32),
        pltpu.VMEM((zr, w_cols), jnp.float32),
        pltpu.VMEM((2, k), jnp.int32),
        pltpu.VMEM((2, k), jnp.int32),
        pltpu.VMEM((2, k), jnp.int32),
        pltpu.VMEM((2, k, w_cols), jnp.float32),
    ]
    if mode in ("den", "full"):
        scratch += [pltpu.VMEM((2, k), jnp.int32),
                    pltpu.VMEM((2, k), jnp.int32),
                    pltpu.VMEM((2, k, 16), jnp.float32),
                    pltpu.VMEM((2, k, 16), jnp.float32)]
    if mode in ("hs", "full"):
        scratch += [pltpu.VMEM((2, k), jnp.int32),
                    pltpu.VMEM((2, k, nb, 16), jnp.float32)]
    if mode == "hs":
        scratch += [pltpu.VMEM((2, k, 8), jnp.float32)]
    n_sem = 8 if mode == "den" else 6
    scratch += [pltpu.SemaphoreType.DMA] * n_sem

    def body(*refs):
        if mode == "den":
            auxs_r, auxd_r, src_r, dst_r, zh_r, out_r, wc_r = refs[:7]
            rest = refs[7:]
        elif mode == "hs":
            hs_r, wc_r, src_r, dst_r, zh_r, out_r = refs[:6]
            rest = refs[6:]
        else:
            hs_r, auxs_r, auxd_r, src_r, dst_r, zh_r, out_r = refs[:7]
            rest = refs[7:]
        acc, zrows, srcb, dstb, dsts, rowsb = rest[:6]
        rest = rest[6:]
        if mode in ("den", "full"):
            esix, edix, esr, edr = rest[:4]
            rest = rest[4:]
        if mode in ("hs", "full"):
            hsix, hsr = rest[:2]
            rest = rest[2:]
        if mode == "hs":
            wrows = rest[0]
            rest = rest[1:]
        sems = rest
        s_idx = sems[0:2]
        s_g = sems[2:4]
        s_sc = sems[4:6]
        s_w = sems[6:8] if mode == "den" else None

        c = lax.axis_index("c")
        s = lax.axis_index("s")
        wid = s * NC + c

        pltpu.sync_copy(zh_r, zrows)
        n_myblk = lax.div(n_blk - 1 - s, NS) + 1

        def zbody(i, carry):
            pltpu.sync_copy(zrows, acc.at[pl.ds((s + i * NS) * zr, zr)])
            return carry
        lax.fori_loop(0, n_myblk, zbody, 0)
        plsc.subcore_barrier()

        iot = lax.iota(jnp.int32, 16)
        n_my = lax.div(n_chunks - 1 - wid, NW) + 1

        def issue_idx(slot, j):
            e0 = (wid + j * NW) * k
            pltpu.async_copy(src_r.at[pl.ds(e0, k)], srcb.at[slot], s_idx[slot])
            pltpu.async_copy(dst_r.at[pl.ds(e0, k)], dstb.at[slot], s_idx[slot])

        issue_idx(0, 0)

        def subiter(b, j):
            ob = 1 - b
            e0 = (wid + j * NW) * k
            pltpu.make_async_copy(src_r.at[pl.ds(0, k)], srcb.at[b], s_idx[b]).wait()
            pltpu.make_async_copy(dst_r.at[pl.ds(0, k)], dstb.at[b], s_idx[b]).wait()

            @pl.when(j >= 2)
            def _():
                pltpu.make_async_copy(rowsb.at[b], acc.at[dsts.at[b]], s_sc[b]).wait()
                if mode == "den":
                    pltpu.make_async_copy(rowsb.at[b], wc_r.at[pl.ds(0, k)], s_w[b]).wait()

            for g in range(k // 16):
                sl = pl.ds(g * 16, 16)
                sv = srcb[b, sl]
                dv = dstb[b, sl]
                dsts[b, sl] = dv
                if mode in ("hs", "full"):
                    hsix[b, sl] = sv * hs_mul + hs_add
                if mode in ("den", "full"):
                    esix[b, sl] = sv * 2
                    edix[b, sl] = dv * 2 + 1

            gd = []
            if mode in ("den", "full"):
                gd.append(pltpu.async_copy(auxs_r.at[esix.at[b]], esr.at[b], s_g[b]))
                gd.append(pltpu.async_copy(auxd_r.at[edix.at[b]], edr.at[b], s_g[b]))
            if mode in ("hs", "full"):
                gd.append(pltpu.async_copy(hs_r.at[hsix.at[b]], hsr.at[b], s_g[b]))
            if mode == "hs":
                gd.append(pltpu.async_copy(wc_r.at[pl.ds(e0, k)], wrows.at[b], s_g[b]))

            @pl.when(j + 1 < n_my)
            def _():
                issue_idx(ob, j + 1)

            for d in gd:
                d.wait()

            for g in range(k // 16):
                rvec = g * 16 + iot
                if mode in ("den", "full"):
                    wv = {}
                    for h in range(H):
                        e_s = plsc.load_gather(
                            esr.at[b], [rvec, jnp.full((16,), es_off + h, jnp.int32)])
                        e_d = plsc.load_gather(
                            edr.at[b], [rvec, jnp.full((16,), ed_off + h, jnp.int32)])
                        e = e_s + e_d
                        e = jnp.where(e > 0, e, 0.2 * e)
                        wv[h] = jnp.exp(e)
                        plsc.store_scatter(
                            rowsb.at[b],
                            [rvec, jnp.full((16,), nb * 16 + h, jnp.int32)],
                            wv[h])
                else:
                    wv = {}
                    for q in range(2):
                        wv[2 * pair + q] = plsc.load_gather(
                            wrows.at[b], [rvec, jnp.full((16,), 2 * pair + q, jnp.int32)])
                for blk in range(nb):
                    wq = wv[2 * pair + blk] if mode == "hs" else wv[blk]
                    for ch in range(16):
                        hv = plsc.load_gather(
                            hsr.at[b],
                            [rvec, jnp.full((16,), blk, jnp.int32),
                             jnp.full((16,), ch, jnp.int32)])
                        plsc.store_scatter(
                            rowsb.at[b],
                            [rvec, jnp.full((16,), blk * 16 + ch, jnp.int32)],
                            hv * wq)

            pltpu.async_copy(rowsb.at[b], acc.at[dsts.at[b]], s_sc[b], add=True)
            if mode == "den":
                pltpu.async_copy(rowsb.at[b], wc_r.at[pl.ds(e0, k)], s_w[b])

        def pairiter(ii, carry):
            for b in range(2):
                j = ii * 2 + b

                @pl.when(j < n_my)
                def _(b=b, j=j):
                    subiter(b, j)
            return carry

        lax.fori_loop(0, lax.div(n_my + 1, 2), pairiter, 0)

        pltpu.make_async_copy(rowsb.at[0], acc.at[dsts.at[0]], s_sc[0]).wait()
        if mode == "den":
            pltpu.make_async_copy(rowsb.at[0], wc_r.at[pl.ds(0, k)], s_w[0]).wait()

        @pl.when(n_my > 1)
        def _():
            pltpu.make_async_copy(rowsb.at[1], acc.at[dsts.at[1]], s_sc[1]).wait()
            if mode == "den":
                pltpu.make_async_copy(rowsb.at[1], wc_r.at[pl.ds(0, k)], s_w[1]).wait()

        plsc.subcore_barrier()

        def wbody(i, carry):
            b0 = (s + i * NS) * zr
            pltpu.sync_copy(acc.at[pl.ds(b0, zr)],
                            out_r.at[pl.ds(c * n_dst + b0, zr)])
            return carry
        lax.fori_loop(0, n_myblk, wbody, 0)

    outs = pl.kernel(
        body,
        out_type=tuple(out_type) if len(out_type) > 1 else out_type[0],
        mesh=mesh,
        compiler_params=pltpu.CompilerParams(needs_layout_passes=False,
                                             use_tc_tiling_on_sc=False),
        scratch_types=scratch,
    )(*operands)
    if mode == "den":
        part, wc = outs
        return part.reshape(2, n_dst, w_cols), wc
    return outs.reshape(2, n_dst, w_cols)


_BN_INV = 1.0 / math.sqrt(1.0 + 1e-5)


def _norm_from_passes(den, ts_list, bias):
    unnorm = jnp.concatenate(list(ts_list), axis=1)
    r = unnorm.shape[0]
    dinv = 1.0 / (den + 1e-16)
    dinvb = jnp.reshape(jnp.broadcast_to(dinv[:, :, None], (r, 8, 16)), (r, 128))
    return unnorm * dinvb + bias


def _epilogue_mol(con_den, con, sim_den, sim, b_con, b_sim, g, bb,
                  block_rows=400):
    def body(cd, c0, c1, c2, c3, sd, s0, s1, s2, s3, bc, bs, g_r, bb_r, o_ref):
        def ts(ref):
            a = ref[...]
            return a[0] + a[1]
        out_c = _norm_from_passes(ts(cd), [ts(c0), ts(c1), ts(c2), ts(c3)], bc[...])
        out_s = _norm_from_passes(ts(sd), [ts(s0), ts(s1), ts(s2), ts(s3)], bs[...])
        x = (out_c + out_s) * _BN_INV * g_r[...] + bb_r[...]
        o_ref[...] = jnp.maximum(x, 0.0)

    specs = []
    for arr in (con_den,) + con + (sim_den,) + sim:
        w = arr.shape[2]
        specs.append(pl.BlockSpec((2, block_rows, w), lambda i: (0, i, 0)))
    for _ in range(4):
        specs.append(pl.BlockSpec((1, 128), lambda i: (0, 0)))
    return pl.pallas_call(
        body,
        grid=(NM // block_rows,),
        in_specs=specs,
        out_specs=pl.BlockSpec((block_rows, 128), lambda i: (i, 0)),
        out_shape=jax.ShapeDtypeStruct((NM, 128), jnp.float32),
    )(con_den, *con, sim_den, *sim, b_con.reshape(1, 128),
      b_sim.reshape(1, 128), g.reshape(1, 128), bb.reshape(1, 128))


def _epilogue_ing(rev, cooc, b_rev, b_cooc, g, bb, block_rows=400):
    def body(r_ref, c_ref, br, bc, g_r, bb_r, o_ref):
        def one(ref, bias):
            a = ref[...]
            ts = a[0] + a[1]
            unnorm = ts[:, :128]
            den = ts[:, 128:136]
            r = unnorm.shape[0]
            dinv = 1.0 / (den + 1e-16)
            dinvb = jnp.reshape(
                jnp.broadcast_to(dinv[:, :, None], (r, 8, 16)), (r, 128))
            return unnorm * dinvb + bias
        x = one(r_ref, br[...]) + one(c_ref, bc[...])
        x = x * _BN_INV * g_r[...] + bb_r[...]
        o_ref[...] = jnp.maximum(x, 0.0)

    return pl.pallas_call(
        body,
        grid=(NI // block_rows,),
        in_specs=[
            pl.BlockSpec((2, block_rows, 136), lambda i: (0, i, 0)),
            pl.BlockSpec((2, block_rows, 136), lambda i: (0, i, 0)),
            pl.BlockSpec((1, 128), lambda i: (0, 0)),
            pl.BlockSpec((1, 128), lambda i: (0, 0)),
            pl.BlockSpec((1, 128), lambda i: (0, 0)),
            pl.BlockSpec((1, 128), lambda i: (0, 0)),
        ],
        out_specs=pl.BlockSpec((block_rows, 128), lambda i: (i, 0)),
        out_shape=jax.ShapeDtypeStruct((NI, 128), jnp.float32),
    )(rev, cooc, b_rev.reshape(1, 128), b_cooc.reshape(1, 128),
      g.reshape(1, 128), bb.reshape(1, 128))


def _fold_att(w_mat, a_vec):
    return (w_mat.reshape(HID, H, C) * a_vec[None]).sum(-1)


def kernel(x_ingredient, x_molecule, ei_contains_src, ei_contains_dst,
           ei_rev_src, ei_rev_dst, ei_cooc_src, ei_cooc_dst,
           ei_sim_src, ei_sim_dst, params):
    p = params
    cs, cd = ei_contains_src, ei_contains_dst
    rs, rd = ei_rev_src, ei_rev_dst
    os_, od_ = ei_cooc_src, ei_cooc_dst
    ss, sd = ei_sim_src, ei_sim_dst

    z8 = jnp.zeros((40, 8), jnp.float32)
    z32 = jnp.zeros((40, 32), jnp.float32)
    z136 = jnp.zeros((16, 136), jnp.float32)

    x_i = _mm(x_ingredient, p["proj_ing_W"], p["proj_ing_b"], relu=True)
    x_m = _mm(x_molecule, p["proj_mol_W"], p["proj_mol_b"], relu=True)

    for l in range(L):
        pre = "l%d_" % l
        ws_con, wd_con = p[pre + "con_Ws"], p[pre + "con_Wd"]
        ws_rev, wd_rev = p[pre + "rev_Ws"], p[pre + "rev_Wd"]
        ws_cooc = p[pre + "cooc_Ws"]
        ws_sim = p[pre + "sim_Ws"]

        w_i_big = jnp.concatenate([ws_con, ws_cooc], axis=1)
        w_m_big = jnp.concatenate([ws_rev, ws_sim], axis=1)
        a_i = jnp.concatenate([
            _fold_att(ws_con, p[pre + "con_as"]),
            _fold_att(ws_cooc, p[pre + "cooc_as"]),
            _fold_att(ws_cooc, p[pre + "cooc_ad"]),
            _fold_att(wd_rev, p[pre + "rev_ad"]),
        ], axis=1)
        a_m = jnp.concatenate([
            _fold_att(ws_rev, p[pre + "rev_as"]),
            _fold_att(ws_sim, p[pre + "sim_as"]),
            _fold_att(ws_sim, p[pre + "sim_ad"]),
            _fold_att(wd_con, p[pre + "con_ad"]),
        ], axis=1)

        h_i = _mm(x_i, w_i_big)
        h_m = _mm(x_m, w_m_big)
        aux_i = _mm(x_i, a_i).reshape(NI * 2, 16)
        aux_m = _mm(x_m, a_m).reshape(NM * 2, 16)

        h_i_pairs = h_i.reshape(NI * 8, 2, 16)
        h_m_pairs = h_m.reshape(NM * 8, 2, 16)
        h_i_full = h_i.reshape(NI * 2, 8, 16)
        h_m_full = h_m.reshape(NM * 2, 8, 16)

        con_den, con_w = _sc_pass(
            "den", (aux_i, aux_m, cs, cd, z8),
            e_total=320000, k=128, n_dst=NM, zr=40, es_off=0, ed_off=8)
        sim_den, sim_w = _sc_pass(
            "den", (aux_m, aux_m, ss, sd, z8),
            e_total=320000, k=128, n_dst=NM, zr=40, es_off=8, ed_off=0)
        con_passes = tuple(
            _sc_pass("hs", (h_i_pairs, con_w, cs, cd, z32),
                     e_total=320000, k=128, n_dst=NM, zr=40,
                     hs_mul=8, hs_add=pr, pair=pr)
            for pr in range(4))
        sim_passes = tuple(
            _sc_pass("hs", (h_m_pairs, sim_w, ss, sd, z32),
                     e_total=320000, k=128, n_dst=NM, zr=40,
                     hs_mul=8, hs_add=4 + pr, pair=pr)
            for pr in range(4))
        rev_pass = _sc_pass(
            "full", (h_m_full, aux_m, aux_i, rs, rd, z136),
            e_total=320000, k=64, n_dst=NI, zr=16,
            hs_mul=2, hs_add=0, es_off=0, ed_off=8)
        cooc_pass = _sc_pass(
            "full", (h_i_full, aux_i, aux_i, os_, od_, z136),
            e_total=160000, k=64, n_dst=NI, zr=16,
            hs_mul=2, hs_add=1, es_off=8, ed_off=0)

        x_m = _epilogue_mol(con_den, con_passes, sim_den, sim_passes,
                            p[pre + "con_b"], p[pre + "sim_b"],
                            p[pre + "bn_mol_g"], p[pre + "bn_mol_b"])
        x_i = _epilogue_ing(rev_pass, cooc_pass,
                            p[pre + "rev_b"], p[pre + "cooc_b"],
                            p[pre + "bn_ing_g"], p[pre + "bn_ing_b"])

    oi = _mm(x_i, p["emb_ing_W"], p["emb_ing_b"])
    om = _mm(x_m, p["emb_mol_W"], p["emb_mol_b"])
    return oi, om


# speedup vs baseline: 52.3801x; 2.4863x over previous
import functools
import math

import jax
import jax.numpy as jnp
from jax import lax
from jax.experimental import pallas as pl
from jax.experimental.pallas import tpu as pltpu
from jax.experimental.pallas import tpu_sc as plsc

HID = 128
H = 8
C = 16
EMB = 64
L = 3
NI = 10000
NM = 50000

NC = 2
NS = 16
NW = NC * NS
ZR = 40


def _mm(x, w, b=None, relu=False, block_rows=512):
    n, kd = x.shape
    co = w.shape[1]
    if b is None:
        b = jnp.zeros((co,), jnp.float32)

    def body(x_ref, w_ref, b_ref, o_ref):
        acc = jnp.dot(x_ref[...], w_ref[...], preferred_element_type=jnp.float32)
        acc = acc + b_ref[...]
        if relu:
            acc = jnp.maximum(acc, 0.0)
        o_ref[...] = acc

    return pl.pallas_call(
        body,
        grid=(pl.cdiv(n, block_rows),),
        in_specs=[
            pl.BlockSpec((block_rows, kd), lambda i: (i, 0)),
            pl.BlockSpec((kd, co), lambda i: (0, 0)),
            pl.BlockSpec((1, co), lambda i: (0, 0)),
        ],
        out_specs=pl.BlockSpec((block_rows, co), lambda i: (i, 0)),
        out_shape=jax.ShapeDtypeStruct((n, co), jnp.float32),
    )(x, w, b.reshape(1, co))


def _sc_pass(mode, operands, *, e_total, k, n_dst, zr, hs_mul=0, hs_add=0,
             es_off=0, ed_off=0, pair=0):
    w_cols = {"den": 8, "hs": 32, "full": 136}[mode]
    nb = {"den": 0, "hs": 2, "full": 8}[mode]
    n_chunks = e_total // k
    n_blk = n_dst // zr
    mesh = plsc.VectorSubcoreMesh(core_axis_name="c", subcore_axis_name="s",
                                  num_cores=NC, num_subcores=NS)

    n_in = len(operands)
    out_type = [jax.ShapeDtypeStruct((2 * n_dst, w_cols), jnp.float32)]
    if mode == "den":
        out_type.append(jax.ShapeDtypeStruct((e_total, 8), jnp.float32))

    scratch = [
        pltpu.VMEM_SHARED((n_dst, w_cols), jnp.float32),
        pltpu.VMEM((zr, w_cols), jnp.float32),
        pltpu.VMEM((2, k), jnp.int32),
        pltpu.VMEM((2, k), jnp.int32),
        pltpu.VMEM((2, k), jnp.int32),
        pltpu.VMEM((2, k, w_cols), jnp.float32),
    ]
    if mode in ("den", "full"):
        scratch += [pltpu.VMEM((2, k), jnp.int32),
                    pltpu.VMEM((2, k), jnp.int32),
                    pltpu.VMEM((2, k, 16), jnp.float32),
                    pltpu.VMEM((2, k, 16), jnp.float32)]
    if mode in ("hs", "full"):
        scratch += [pltpu.VMEM((2, k), jnp.int32),
                    pltpu.VMEM((2, k, nb, 16), jnp.float32)]
    if mode == "hs":
        scratch += [pltpu.VMEM((2, k // 2, 16), jnp.float32)]
    n_sem = 8 if mode == "den" else 6
    scratch += [pltpu.SemaphoreType.DMA] * n_sem

    def body(*refs):
        if mode == "den":
            auxs_r, auxd_r, src_r, dst_r, zh_r, out_r, wc_r = refs[:7]
            rest = refs[7:]
        elif mode == "hs":
            hs_r, wc_r, src_r, dst_r, zh_r, out_r = refs[:6]
            rest = refs[6:]
        else:
            hs_r, auxs_r, auxd_r, src_r, dst_r, zh_r, out_r = refs[:7]
            rest = refs[7:]
        acc, zrows, srcb, dstb, dsts, rowsb = rest[:6]
        rest = rest[6:]
        if mode in ("den", "full"):
            esix, edix, esr, edr = rest[:4]
            rest = rest[4:]
        if mode in ("hs", "full"):
            hsix, hsr = rest[:2]
            rest = rest[2:]
        if mode == "hs":
            wrows = rest[0]
            rest = rest[1:]
        sems = rest
        s_idx = sems[0:2]
        s_g = sems[2:4]
        s_sc = sems[4:6]
        s_w = sems[6:8] if mode == "den" else None

        c = lax.axis_index("c")
        s = lax.axis_index("s")
        wid = s * NC + c

        pltpu.sync_copy(zh_r, zrows)
        n_myblk = lax.div(n_blk - 1 - s, NS) + 1

        def zbody(i, carry):
            pltpu.sync_copy(zrows, acc.at[pl.ds((s + i * NS) * zr, zr)])
            return carry
        lax.fori_loop(0, n_myblk, zbody, 0)
        plsc.subcore_barrier()

        iot = lax.iota(jnp.int32, 16)
        n_my = lax.div(n_chunks - 1 - wid, NW) + 1

        def issue_idx(slot, j):
            e0 = (wid + j * NW) * k
            pltpu.async_copy(src_r.at[pl.ds(e0, k)], srcb.at[slot], s_idx[slot])
            pltpu.async_copy(dst_r.at[pl.ds(e0, k)], dstb.at[slot], s_idx[slot])

        issue_idx(0, 0)

        def subiter(b, j):
            ob = 1 - b
            e0 = (wid + j * NW) * k
            pltpu.make_async_copy(src_r.at[pl.ds(0, k)], srcb.at[b], s_idx[b]).wait()
            pltpu.make_async_copy(dst_r.at[pl.ds(0, k)], dstb.at[b], s_idx[b]).wait()

            @pl.when(j >= 2)
            def _():
                pltpu.make_async_copy(rowsb.at[b], acc.at[dsts.at[b]], s_sc[b]).wait()
                if mode == "den":
                    pltpu.make_async_copy(rowsb.at[b], wc_r.at[pl.ds(0, k)], s_w[b]).wait()

            for g in range(k // 16):
                sl = pl.ds(g * 16, 16)
                sv = srcb[b, sl]
                dv = dstb[b, sl]
                dsts[b, sl] = dv
                if mode in ("hs", "full"):
                    hsix[b, sl] = sv * hs_mul + hs_add
                if mode in ("den", "full"):
                    esix[b, sl] = sv * 2
                    edix[b, sl] = dv * 2 + 1

            gd = []
            if mode in ("den", "full"):
                gd.append(pltpu.async_copy(auxs_r.at[esix.at[b]], esr.at[b], s_g[b]))
                gd.append(pltpu.async_copy(auxd_r.at[edix.at[b]], edr.at[b], s_g[b]))
            if mode in ("hs", "full"):
                gd.append(pltpu.async_copy(hs_r.at[hsix.at[b]], hsr.at[b], s_g[b]))
            if mode == "hs":
                gd.append(pltpu.async_copy(
                    wc_r.at[pl.ds((wid + j * NW) * (k // 2), k // 2)],
                    wrows.at[b], s_g[b]))

            @pl.when(j + 1 < n_my)
            def _():
                issue_idx(ob, j + 1)

            for d in gd:
                d.wait()

            wlist = []
            if mode in ("den", "full"):
                for g in range(k // 16):
                    rvec = g * 16 + iot
                    wg = []
                    for h in range(H):
                        e_s = plsc.load_gather(
                            esr.at[b], [rvec, jnp.full((16,), es_off + h, jnp.int32)])
                        e_d = plsc.load_gather(
                            edr.at[b], [rvec, jnp.full((16,), ed_off + h, jnp.int32)])
                        e = e_s + e_d
                        e = jnp.where(e > 0, e, 0.2 * e)
                        w = jnp.exp(e)
                        wg.append(w)
                        plsc.store_scatter(
                            rowsb.at[b],
                            [rvec, jnp.full((16,), nb * 16 + h, jnp.int32)],
                            w)
                    wlist.append(wg)
            if mode == "full":
                for e_i in range(k):
                    g, lane = divmod(e_i, 16)
                    for blk in range(nb):
                        w_s = wlist[g][blk][lane]
                        rowsb[b, e_i, pl.ds(blk * 16, 16)] = (
                            hsr[b, e_i, blk, :] * w_s)
            if mode == "hs":
                for e2 in range(k // 2):
                    wvec = wrows[b, e2, :]
                    for half in range(2):
                        e_i = 2 * e2 + half
                        for blk in range(2):
                            w_s = wvec[half * 8 + 2 * pair + blk]
                            rowsb[b, e_i, pl.ds(blk * 16, 16)] = (
                                hsr[b, e_i, blk, :] * w_s)

            pltpu.async_copy(rowsb.at[b], acc.at[dsts.at[b]], s_sc[b], add=True)
            if mode == "den":
                pltpu.async_copy(rowsb.at[b], wc_r.at[pl.ds(e0, k)], s_w[b])

        def pairiter(ii, carry):
            for b in range(2):
                j = ii * 2 + b

                @pl.when(j < n_my)
                def _(b=b, j=j):
                    subiter(b, j)
            return carry

        lax.fori_loop(0, lax.div(n_my + 1, 2), pairiter, 0)

        pltpu.make_async_copy(rowsb.at[0], acc.at[dsts.at[0]], s_sc[0]).wait()
        if mode == "den":
            pltpu.make_async_copy(rowsb.at[0], wc_r.at[pl.ds(0, k)], s_w[0]).wait()

        @pl.when(n_my > 1)
        def _():
            pltpu.make_async_copy(rowsb.at[1], acc.at[dsts.at[1]], s_sc[1]).wait()
            if mode == "den":
                pltpu.make_async_copy(rowsb.at[1], wc_r.at[pl.ds(0, k)], s_w[1]).wait()

        plsc.subcore_barrier()

        def wbody(i, carry):
            b0 = (s + i * NS) * zr
            pltpu.sync_copy(acc.at[pl.ds(b0, zr)],
                            out_r.at[pl.ds(c * n_dst + b0, zr)])
            return carry
        lax.fori_loop(0, n_myblk, wbody, 0)

    outs = pl.kernel(
        body,
        out_type=tuple(out_type) if len(out_type) > 1 else out_type[0],
        mesh=mesh,
        compiler_params=pltpu.CompilerParams(needs_layout_passes=False,
                                             use_tc_tiling_on_sc=False),
        scratch_types=scratch,
    )(*operands)
    if mode == "den":
        part, wc = outs
        return part.reshape(2, n_dst, w_cols), wc
    return outs.reshape(2, n_dst, w_cols)


_BN_INV = 1.0 / math.sqrt(1.0 + 1e-5)


def _norm_from_passes(den, ts_list, bias):
    unnorm = jnp.concatenate(list(ts_list), axis=1)
    r = unnorm.shape[0]
    dinv = 1.0 / (den + 1e-16)
    dinvb = jnp.reshape(jnp.broadcast_to(dinv[:, :, None], (r, 8, 16)), (r, 128))
    return unnorm * dinvb + bias


def _epilogue_mol(con_den, con, sim_den, sim, b_con, b_sim, g, bb,
                  block_rows=400):
    def body(cd, c0, c1, c2, c3, sd, s0, s1, s2, s3, bc, bs, g_r, bb_r, o_ref):
        def ts(ref):
            a = ref[...]
            return a[0] + a[1]
        out_c = _norm_from_passes(ts(cd), [ts(c0), ts(c1), ts(c2), ts(c3)], bc[...])
        out_s = _norm_from_passes(ts(sd), [ts(s0), ts(s1), ts(s2), ts(s3)], bs[...])
        x = (out_c + out_s) * _BN_INV * g_r[...] + bb_r[...]
        o_ref[...] = jnp.maximum(x, 0.0)

    specs = []
    for arr in (con_den,) + con + (sim_den,) + sim:
        w = arr.shape[2]
        specs.append(pl.BlockSpec((2, block_rows, w), lambda i: (0, i, 0)))
    for _ in range(4):
        specs.append(pl.BlockSpec((1, 128), lambda i: (0, 0)))
    return pl.pallas_call(
        body,
        grid=(NM // block_rows,),
        in_specs=specs,
        out_specs=pl.BlockSpec((block_rows, 128), lambda i: (i, 0)),
        out_shape=jax.ShapeDtypeStruct((NM, 128), jnp.float32),
    )(con_den, *con, sim_den, *sim, b_con.reshape(1, 128),
      b_sim.reshape(1, 128), g.reshape(1, 128), bb.reshape(1, 128))


def _epilogue_ing(rev, cooc, b_rev, b_cooc, g, bb, block_rows=400):
    def body(r_ref, c_ref, br, bc, g_r, bb_r, o_ref):
        def one(ref, bias):
            a = ref[...]
            ts = a[0] + a[1]
            unnorm = ts[:, :128]
            den = ts[:, 128:136]
            r = unnorm.shape[0]
            dinv = 1.0 / (den + 1e-16)
            dinvb = jnp.reshape(
                jnp.broadcast_to(dinv[:, :, None], (r, 8, 16)), (r, 128))
            return unnorm * dinvb + bias
        x = one(r_ref, br[...]) + one(c_ref, bc[...])
        x = x * _BN_INV * g_r[...] + bb_r[...]
        o_ref[...] = jnp.maximum(x, 0.0)

    return pl.pallas_call(
        body,
        grid=(NI // block_rows,),
        in_specs=[
            pl.BlockSpec((2, block_rows, 136), lambda i: (0, i, 0)),
            pl.BlockSpec((2, block_rows, 136), lambda i: (0, i, 0)),
            pl.BlockSpec((1, 128), lambda i: (0, 0)),
            pl.BlockSpec((1, 128), lambda i: (0, 0)),
            pl.BlockSpec((1, 128), lambda i: (0, 0)),
            pl.BlockSpec((1, 128), lambda i: (0, 0)),
        ],
        out_specs=pl.BlockSpec((block_rows, 128), lambda i: (i, 0)),
        out_shape=jax.ShapeDtypeStruct((NI, 128), jnp.float32),
    )(rev, cooc, b_rev.reshape(1, 128), b_cooc.reshape(1, 128),
      g.reshape(1, 128), bb.reshape(1, 128))


def _fold_att(w_mat, a_vec):
    return (w_mat.reshape(HID, H, C) * a_vec[None]).sum(-1)


def kernel(x_ingredient, x_molecule, ei_contains_src, ei_contains_dst,
           ei_rev_src, ei_rev_dst, ei_cooc_src, ei_cooc_dst,
           ei_sim_src, ei_sim_dst, params):
    p = params
    cs, cd = ei_contains_src, ei_contains_dst
    rs, rd = ei_rev_src, ei_rev_dst
    os_, od_ = ei_cooc_src, ei_cooc_dst
    ss, sd = ei_sim_src, ei_sim_dst

    z8 = jnp.zeros((40, 8), jnp.float32)
    z32 = jnp.zeros((40, 32), jnp.float32)
    z136 = jnp.zeros((16, 136), jnp.float32)

    x_i = _mm(x_ingredient, p["proj_ing_W"], p["proj_ing_b"], relu=True)
    x_m = _mm(x_molecule, p["proj_mol_W"], p["proj_mol_b"], relu=True)

    for l in range(L):
        pre = "l%d_" % l
        ws_con, wd_con = p[pre + "con_Ws"], p[pre + "con_Wd"]
        ws_rev, wd_rev = p[pre + "rev_Ws"], p[pre + "rev_Wd"]
        ws_cooc = p[pre + "cooc_Ws"]
        ws_sim = p[pre + "sim_Ws"]

        w_i_big = jnp.concatenate([ws_con, ws_cooc], axis=1)
        w_m_big = jnp.concatenate([ws_rev, ws_sim], axis=1)
        a_i = jnp.concatenate([
            _fold_att(ws_con, p[pre + "con_as"]),
            _fold_att(ws_cooc, p[pre + "cooc_as"]),
            _fold_att(ws_cooc, p[pre + "cooc_ad"]),
            _fold_att(wd_rev, p[pre + "rev_ad"]),
        ], axis=1)
        a_m = jnp.concatenate([
            _fold_att(ws_rev, p[pre + "rev_as"]),
            _fold_att(ws_sim, p[pre + "sim_as"]),
            _fold_att(ws_sim, p[pre + "sim_ad"]),
            _fold_att(wd_con, p[pre + "con_ad"]),
        ], axis=1)

        h_i = _mm(x_i, w_i_big)
        h_m = _mm(x_m, w_m_big)
        aux_i = _mm(x_i, a_i).reshape(NI * 2, 16)
        aux_m = _mm(x_m, a_m).reshape(NM * 2, 16)

        h_i_pairs = h_i.reshape(NI * 8, 2, 16)
        h_m_pairs = h_m.reshape(NM * 8, 2, 16)
        h_i_full = h_i.reshape(NI * 2, 8, 16)
        h_m_full = h_m.reshape(NM * 2, 8, 16)

        con_den, con_w = _sc_pass(
            "den", (aux_i, aux_m, cs, cd, z8),
            e_total=320000, k=128, n_dst=NM, zr=40, es_off=0, ed_off=8)
        sim_den, sim_w = _sc_pass(
            "den", (aux_m, aux_m, ss, sd, z8),
            e_total=320000, k=128, n_dst=NM, zr=40, es_off=8, ed_off=0)
        con_w2 = con_w.reshape(160000, 16)
        sim_w2 = sim_w.reshape(160000, 16)
        con_passes = tuple(
            _sc_pass("hs", (h_i_pairs, con_w2, cs, cd, z32),
                     e_total=320000, k=128, n_dst=NM, zr=40,
                     hs_mul=8, hs_add=pr, pair=pr)
            for pr in range(4))
        sim_passes = tuple(
            _sc_pass("hs", (h_m_pairs, sim_w2, ss, sd, z32),
                     e_total=320000, k=128, n_dst=NM, zr=40,
                     hs_mul=8, hs_add=4 + pr, pair=pr)
            for pr in range(4))
        rev_pass = _sc_pass(
            "full", (h_m_full, aux_m, aux_i, rs, rd, z136),
            e_total=320000, k=64, n_dst=NI, zr=16,
            hs_mul=2, hs_add=0, es_off=0, ed_off=8)
        cooc_pass = _sc_pass(
            "full", (h_i_full, aux_i, aux_i, os_, od_, z136),
            e_total=160000, k=64, n_dst=NI, zr=16,
            hs_mul=2, hs_add=1, es_off=8, ed_off=0)

        x_m = _epilogue_mol(con_den, con_passes, sim_den, sim_passes,
                            p[pre + "con_b"], p[pre + "sim_b"],
                            p[pre + "bn_mol_g"], p[pre + "bn_mol_b"])
        x_i = _epilogue_ing(rev_pass, cooc_pass,
                            p[pre + "rev_b"], p[pre + "cooc_b"],
                            p[pre + "bn_ing_g"], p[pre + "bn_ing_b"])

    oi = _mm(x_i, p["emb_ing_W"], p["emb_ing_b"])
    om = _mm(x_m, p["emb_mol_W"], p["emb_mol_b"])
    return oi, om


# speedup vs baseline: 62.8642x; 1.2002x over previous
import functools
import math

import jax
import jax.numpy as jnp
from jax import lax
from jax.experimental import pallas as pl
from jax.experimental.pallas import tpu as pltpu
from jax.experimental.pallas import tpu_sc as plsc

HID = 128
H = 8
C = 16
EMB = 64
L = 3
NI = 10000
NM = 50000

NC = 2
NS = 16
NW = NC * NS
ZR = 40


def _mm(x, w, b=None, relu=False, block_rows=512):
    n, kd = x.shape
    co = w.shape[1]
    if b is None:
        b = jnp.zeros((co,), jnp.float32)

    def body(x_ref, w_ref, b_ref, o_ref):
        acc = jnp.dot(x_ref[...], w_ref[...], preferred_element_type=jnp.float32)
        acc = acc + b_ref[...]
        if relu:
            acc = jnp.maximum(acc, 0.0)
        o_ref[...] = acc

    return pl.pallas_call(
        body,
        grid=(pl.cdiv(n, block_rows),),
        in_specs=[
            pl.BlockSpec((block_rows, kd), lambda i: (i, 0)),
            pl.BlockSpec((kd, co), lambda i: (0, 0)),
            pl.BlockSpec((1, co), lambda i: (0, 0)),
        ],
        out_specs=pl.BlockSpec((block_rows, co), lambda i: (i, 0)),
        out_shape=jax.ShapeDtypeStruct((n, co), jnp.float32),
    )(x, w, b.reshape(1, co))


def _sc_pass(mode, operands, *, e_total, k, n_dst, zr, hs_mul=0, hs_add=0,
             es_off=0, ed_off=0, pair=0):
    w_cols = {"den": 8, "hs": 32, "full": 136}[mode]
    nb = {"den": 0, "hs": 2, "full": 8}[mode]
    n_chunks = e_total // k
    n_blk = n_dst // zr
    mesh = plsc.VectorSubcoreMesh(core_axis_name="c", subcore_axis_name="s",
                                  num_cores=NC, num_subcores=NS)

    n_in = len(operands)
    out_type = [jax.ShapeDtypeStruct((2 * n_dst, w_cols), jnp.float32)]
    if mode == "den":
        out_type.append(jax.ShapeDtypeStruct((e_total, 8), jnp.float32))

    scratch = [
        pltpu.VMEM_SHARED((n_dst, w_cols), jnp.float32),
        pltpu.VMEM((zr, w_cols), jnp.float32),
        pltpu.VMEM((2, k), jnp.int32),
        pltpu.VMEM((2, k), jnp.int32),
        pltpu.VMEM((4, k), jnp.int32),
        pltpu.VMEM((4, k, w_cols), jnp.float32),
    ]
    if mode in ("den", "full"):
        scratch += [pltpu.VMEM((2, k), jnp.int32),
                    pltpu.VMEM((2, k), jnp.int32),
                    pltpu.VMEM((2, k, 16), jnp.float32),
                    pltpu.VMEM((2, k, 16), jnp.float32)]
    if mode in ("hs", "full"):
        scratch += [pltpu.VMEM((2, k), jnp.int32),
                    pltpu.VMEM((2, k, nb, 16), jnp.float32)]
    if mode == "hs":
        scratch += [pltpu.VMEM((2, k // 2, 16), jnp.float32)]
    n_sem = 12 if mode == "den" else 8
    scratch += [pltpu.SemaphoreType.DMA] * n_sem

    def body(*refs):
        if mode == "den":
            auxs_r, auxd_r, src_r, dst_r, zh_r, out_r, wc_r = refs[:7]
            rest = refs[7:]
        elif mode == "hs":
            hs_r, wc_r, src_r, dst_r, zh_r, out_r = refs[:6]
            rest = refs[6:]
        else:
            hs_r, auxs_r, auxd_r, src_r, dst_r, zh_r, out_r = refs[:7]
            rest = refs[7:]
        acc, zrows, srcb, dstb, dsts, rowsb = rest[:6]
        rest = rest[6:]
        if mode in ("den", "full"):
            esix, edix, esr, edr = rest[:4]
            rest = rest[4:]
        if mode in ("hs", "full"):
            hsix, hsr = rest[:2]
            rest = rest[2:]
        if mode == "hs":
            wrows = rest[0]
            rest = rest[1:]
        sems = rest
        s_idx = sems[0:2]
        s_g = sems[2:4]
        s_sc = sems[4:8]
        s_w = sems[8:12] if mode == "den" else None

        c = lax.axis_index("c")
        s = lax.axis_index("s")
        wid = s * NC + c

        pltpu.sync_copy(zh_r, zrows)
        n_myblk = lax.div(n_blk - 1 - s, NS) + 1

        def zbody(i, carry):
            pltpu.sync_copy(zrows, acc.at[pl.ds((s + i * NS) * zr, zr)])
            return carry
        lax.fori_loop(0, n_myblk, zbody, 0)
        plsc.subcore_barrier()

        iot = lax.iota(jnp.int32, 16)
        n_my = lax.div(n_chunks - 1 - wid, NW) + 1

        def issue_idx(slot, j):
            e0 = (wid + j * NW) * k
            pltpu.async_copy(src_r.at[pl.ds(e0, k)], srcb.at[slot], s_idx[slot])
            pltpu.async_copy(dst_r.at[pl.ds(e0, k)], dstb.at[slot], s_idx[slot])

        def drain_scatter(rb):
            pltpu.make_async_copy(rowsb.at[rb], acc.at[dsts.at[rb]], s_sc[rb]).wait()
            if mode == "den":
                pltpu.make_async_copy(rowsb.at[rb], wc_r.at[pl.ds(0, k)], s_w[rb]).wait()

        def build_and_fetch(j, gb, rb, drain):
            pltpu.make_async_copy(src_r.at[pl.ds(0, k)], srcb.at[gb], s_idx[gb]).wait()
            pltpu.make_async_copy(dst_r.at[pl.ds(0, k)], dstb.at[gb], s_idx[gb]).wait()
            if drain:
                @pl.when(j >= 4)
                def _():
                    drain_scatter(rb)
            for g in range(k // 16):
                sl = pl.ds(g * 16, 16)
                sv = srcb[gb, sl]
                dv = dstb[gb, sl]
                dsts[rb, sl] = dv
                if mode in ("hs", "full"):
                    hsix[gb, sl] = sv * hs_mul + hs_add
                if mode in ("den", "full"):
                    esix[gb, sl] = sv * 2
                    edix[gb, sl] = dv * 2 + 1
            if mode in ("den", "full"):
                pltpu.async_copy(auxs_r.at[esix.at[gb]], esr.at[gb], s_g[gb])
                pltpu.async_copy(auxd_r.at[edix.at[gb]], edr.at[gb], s_g[gb])
            if mode in ("hs", "full"):
                pltpu.async_copy(hs_r.at[hsix.at[gb]], hsr.at[gb], s_g[gb])
            if mode == "hs":
                pltpu.async_copy(
                    wc_r.at[pl.ds((wid + j * NW) * (k // 2), k // 2)],
                    wrows.at[gb], s_g[gb])

        def wait_gathers(gb):
            if mode in ("den", "full"):
                pltpu.make_async_copy(auxs_r.at[esix.at[gb]], esr.at[gb], s_g[gb]).wait()
                pltpu.make_async_copy(auxd_r.at[edix.at[gb]], edr.at[gb], s_g[gb]).wait()
            if mode in ("hs", "full"):
                pltpu.make_async_copy(hs_r.at[hsix.at[gb]], hsr.at[gb], s_g[gb]).wait()
            if mode == "hs":
                pltpu.make_async_copy(
                    wc_r.at[pl.ds(0, k // 2)], wrows.at[gb], s_g[gb]).wait()

        issue_idx(0, 0)
        build_and_fetch(0, 0, 0, drain=False)

        @pl.when(jnp.int32(1) < n_my)
        def _():
            issue_idx(1, 1)

        def subiter(b, j):
            rb = b
            gb = b % 2
            e0 = (wid + j * NW) * k

            @pl.when(j + 1 < n_my)
            def _():
                build_and_fetch(j + 1, (b + 1) % 2, (b + 1) % 4, drain=True)

                @pl.when(j + 2 < n_my)
                def _():
                    issue_idx(b % 2, j + 2)

            wait_gathers(gb)

            wlist = []
            if mode in ("den", "full"):
                for g in range(k // 16):
                    rvec = g * 16 + iot
                    wg = []
                    for h in range(H):
                        e_s = plsc.load_gather(
                            esr.at[gb], [rvec, jnp.full((16,), es_off + h, jnp.int32)])
                        e_d = plsc.load_gather(
                            edr.at[gb], [rvec, jnp.full((16,), ed_off + h, jnp.int32)])
                        e = e_s + e_d
                        e = jnp.where(e > 0, e, 0.2 * e)
                        w = jnp.exp(e)
                        wg.append(w)
                        plsc.store_scatter(
                            rowsb.at[rb],
                            [rvec, jnp.full((16,), nb * 16 + h, jnp.int32)],
                            w)
                    wlist.append(wg)
            if mode == "full":
                for e_i in range(k):
                    g, lane = divmod(e_i, 16)
                    for blk in range(nb):
                        w_s = wlist[g][blk][lane]
                        rowsb[rb, e_i, pl.ds(blk * 16, 16)] = (
                            hsr[gb, e_i, blk, :] * w_s)
            if mode == "hs":
                for e2 in range(k // 2):
                    wvec = wrows[gb, e2, :]
                    for half in range(2):
                        e_i = 2 * e2 + half
                        for blk in range(2):
                            w_s = wvec[half * 8 + 2 * pair + blk]
                            rowsb[rb, e_i, pl.ds(blk * 16, 16)] = (
                                hsr[gb, e_i, blk, :] * w_s)

            pltpu.async_copy(rowsb.at[rb], acc.at[dsts.at[rb]], s_sc[rb], add=True)
            if mode == "den":
                pltpu.async_copy(rowsb.at[rb], wc_r.at[pl.ds(e0, k)], s_w[rb])

        def quaditer(ii, carry):
            for b in range(4):
                j = ii * 4 + b

                @pl.when(j < n_my)
                def _(b=b, j=j):
                    subiter(b, j)
            return carry

        lax.fori_loop(0, lax.div(n_my + 3, 4), quaditer, 0)

        for x in range(4):
            @pl.when(n_my > x)
            def _(x=x):
                drain_scatter(x)

        plsc.subcore_barrier()

        def wbody(i, carry):
            b0 = (s + i * NS) * zr
            pltpu.sync_copy(acc.at[pl.ds(b0, zr)],
                            out_r.at[pl.ds(c * n_dst + b0, zr)])
            return carry
        lax.fori_loop(0, n_myblk, wbody, 0)

    outs = pl.kernel(
        body,
        out_type=tuple(out_type) if len(out_type) > 1 else out_type[0],
        mesh=mesh,
        compiler_params=pltpu.CompilerParams(needs_layout_passes=False,
                                             use_tc_tiling_on_sc=False),
        scratch_types=scratch,
    )(*operands)
    if mode == "den":
        part, wc = outs
        return part.reshape(2, n_dst, w_cols), wc
    return outs.reshape(2, n_dst, w_cols)


_BN_INV = 1.0 / math.sqrt(1.0 + 1e-5)


def _norm_from_passes(den, ts_list, bias):
    unnorm = jnp.concatenate(list(ts_list), axis=1)
    r = unnorm.shape[0]
    dinv = 1.0 / (den + 1e-16)
    dinvb = jnp.reshape(jnp.broadcast_to(dinv[:, :, None], (r, 8, 16)), (r, 128))
    return unnorm * dinvb + bias


def _epilogue_mol(con_den, con, sim_den, sim, b_con, b_sim, g, bb,
                  block_rows=400):
    def body(cd, c0, c1, c2, c3, sd, s0, s1, s2, s3, bc, bs, g_r, bb_r, o_ref):
        def ts(ref):
            a = ref[...]
            return a[0] + a[1]
        out_c = _norm_from_passes(ts(cd), [ts(c0), ts(c1), ts(c2), ts(c3)], bc[...])
        out_s = _norm_from_passes(ts(sd), [ts(s0), ts(s1), ts(s2), ts(s3)], bs[...])
        x = (out_c + out_s) * _BN_INV * g_r[...] + bb_r[...]
        o_ref[...] = jnp.maximum(x, 0.0)

    specs = []
    for arr in (con_den,) + con + (sim_den,) + sim:
        w = arr.shape[2]
        specs.append(pl.BlockSpec((2, block_rows, w), lambda i: (0, i, 0)))
    for _ in range(4):
        specs.append(pl.BlockSpec((1, 128), lambda i: (0, 0)))
    return pl.pallas_call(
        body,
        grid=(NM // block_rows,),
        in_specs=specs,
        out_specs=pl.BlockSpec((block_rows, 128), lambda i: (i, 0)),
        out_shape=jax.ShapeDtypeStruct((NM, 128), jnp.float32),
    )(con_den, *con, sim_den, *sim, b_con.reshape(1, 128),
      b_sim.reshape(1, 128), g.reshape(1, 128), bb.reshape(1, 128))


def _epilogue_ing(rev, cooc, b_rev, b_cooc, g, bb, block_rows=400):
    def body(r_ref, c_ref, br, bc, g_r, bb_r, o_ref):
        def one(ref, bias):
            a = ref[...]
            ts = a[0] + a[1]
            unnorm = ts[:, :128]
            den = ts[:, 128:136]
            r = unnorm.shape[0]
            dinv = 1.0 / (den + 1e-16)
            dinvb = jnp.reshape(
                jnp.broadcast_to(dinv[:, :, None], (r, 8, 16)), (r, 128))
            return unnorm * dinvb + bias
        x = one(r_ref, br[...]) + one(c_ref, bc[...])
        x = x * _BN_INV * g_r[...] + bb_r[...]
        o_ref[...] = jnp.maximum(x, 0.0)

    return pl.pallas_call(
        body,
        grid=(NI // block_rows,),
        in_specs=[
            pl.BlockSpec((2, block_rows, 136), lambda i: (0, i, 0)),
            pl.BlockSpec((2, block_rows, 136), lambda i: (0, i, 0)),
            pl.BlockSpec((1, 128), lambda i: (0, 0)),
            pl.BlockSpec((1, 128), lambda i: (0, 0)),
            pl.BlockSpec((1, 128), lambda i: (0, 0)),
            pl.BlockSpec((1, 128), lambda i: (0, 0)),
        ],
        out_specs=pl.BlockSpec((block_rows, 128), lambda i: (i, 0)),
        out_shape=jax.ShapeDtypeStruct((NI, 128), jnp.float32),
    )(rev, cooc, b_rev.reshape(1, 128), b_cooc.reshape(1, 128),
      g.reshape(1, 128), bb.reshape(1, 128))


def _fold_att(w_mat, a_vec):
    return (w_mat.reshape(HID, H, C) * a_vec[None]).sum(-1)


def kernel(x_ingredient, x_molecule, ei_contains_src, ei_contains_dst,
           ei_rev_src, ei_rev_dst, ei_cooc_src, ei_cooc_dst,
           ei_sim_src, ei_sim_dst, params):
    p = params
    cs, cd = ei_contains_src, ei_contains_dst
    rs, rd = ei_rev_src, ei_rev_dst
    os_, od_ = ei_cooc_src, ei_cooc_dst
    ss, sd = ei_sim_src, ei_sim_dst

    z8 = jnp.zeros((40, 8), jnp.float32)
    z32 = jnp.zeros((40, 32), jnp.float32)
    z136 = jnp.zeros((16, 136), jnp.float32)

    x_i = _mm(x_ingredient, p["proj_ing_W"], p["proj_ing_b"], relu=True)
    x_m = _mm(x_molecule, p["proj_mol_W"], p["proj_mol_b"], relu=True)

    for l in range(L):
        pre = "l%d_" % l
        ws_con, wd_con = p[pre + "con_Ws"], p[pre + "con_Wd"]
        ws_rev, wd_rev = p[pre + "rev_Ws"], p[pre + "rev_Wd"]
        ws_cooc = p[pre + "cooc_Ws"]
        ws_sim = p[pre + "sim_Ws"]

        w_i_big = jnp.concatenate([ws_con, ws_cooc], axis=1)
        w_m_big = jnp.concatenate([ws_rev, ws_sim], axis=1)
        a_i = jnp.concatenate([
            _fold_att(ws_con, p[pre + "con_as"]),
            _fold_att(ws_cooc, p[pre + "cooc_as"]),
            _fold_att(ws_cooc, p[pre + "cooc_ad"]),
            _fold_att(wd_rev, p[pre + "rev_ad"]),
        ], axis=1)
        a_m = jnp.concatenate([
            _fold_att(ws_rev, p[pre + "rev_as"]),
            _fold_att(ws_sim, p[pre + "sim_as"]),
            _fold_att(ws_sim, p[pre + "sim_ad"]),
            _fold_att(wd_con, p[pre + "con_ad"]),
        ], axis=1)

        h_i = _mm(x_i, w_i_big)
        h_m = _mm(x_m, w_m_big)
        aux_i = _mm(x_i, a_i).reshape(NI * 2, 16)
        aux_m = _mm(x_m, a_m).reshape(NM * 2, 16)

        h_i_pairs = h_i.reshape(NI * 8, 2, 16)
        h_m_pairs = h_m.reshape(NM * 8, 2, 16)
        h_i_full = h_i.reshape(NI * 2, 8, 16)
        h_m_full = h_m.reshape(NM * 2, 8, 16)

        con_den, con_w = _sc_pass(
            "den", (aux_i, aux_m, cs, cd, z8),
            e_total=320000, k=128, n_dst=NM, zr=40, es_off=0, ed_off=8)
        sim_den, sim_w = _sc_pass(
            "den", (aux_m, aux_m, ss, sd, z8),
            e_total=320000, k=128, n_dst=NM, zr=40, es_off=8, ed_off=0)
        con_w2 = con_w.reshape(160000, 16)
        sim_w2 = sim_w.reshape(160000, 16)
        con_passes = tuple(
            _sc_pass("hs", (h_i_pairs, con_w2, cs, cd, z32),
                     e_total=320000, k=128, n_dst=NM, zr=40,
                     hs_mul=8, hs_add=pr, pair=pr)
            for pr in range(4))
        sim_passes = tuple(
            _sc_pass("hs", (h_m_pairs, sim_w2, ss, sd, z32),
                     e_total=320000, k=128, n_dst=NM, zr=40,
                     hs_mul=8, hs_add=4 + pr, pair=pr)
            for pr in range(4))
        rev_pass = _sc_pass(
            "full", (h_m_full, aux_m, aux_i, rs, rd, z136),
            e_total=320000, k=32, n_dst=NI, zr=16,
            hs_mul=2, hs_add=0, es_off=0, ed_off=8)
        cooc_pass = _sc_pass(
            "full", (h_i_full, aux_i, aux_i, os_, od_, z136),
            e_total=160000, k=32, n_dst=NI, zr=16,
            hs_mul=2, hs_add=1, es_off=8, ed_off=0)

        x_m = _epilogue_mol(con_den, con_passes, sim_den, sim_passes,
                            p[pre + "con_b"], p[pre + "sim_b"],
                            p[pre + "bn_mol_g"], p[pre + "bn_mol_b"])
        x_i = _epilogue_ing(rev_pass, cooc_pass,
                            p[pre + "rev_b"], p[pre + "cooc_b"],
                            p[pre + "bn_ing_g"], p[pre + "bn_ing_b"])

    oi = _mm(x_i, p["emb_ing_W"], p["emb_ing_b"])
    om = _mm(x_m, p["emb_mol_W"], p["emb_mol_b"])
    return oi, om
